# initial kernel scaffold (unmeasured)
import jax
import jax.numpy as jnp
from jax import lax
from jax.experimental import pallas as pl
from jax.experimental.pallas import tpu as pltpu

N_DEV = 4
M = 1024
K = 1024
N = 8192
NT = 512
N_NT = N // NT
QT = 1024
N_QT = N // QT

F32 = jnp.float32
BF16 = jnp.bfloat16


def kernel(x, w_mat):
    def body(x_ref, w_ref, out_ref, acc, xblk, xsend, xstage, wbuf, wbf,
             amx_me, amx_peers, xsend_sem, xrecv_sem, asend_sem, arecv_sem,
             stage_sem, w_sem, out_sem):
        me = lax.axis_index("i")

        send_rdmas = []
        for d in (1, 3, 2):
            t = lax.rem(me + d, N_DEV)
            cp = pltpu.make_async_copy(
                x_ref.at[pl.ds(t * M, M), :], xstage, stage_sem)
            cp.start()
            cp.wait()
            xsend[d - 1, :, :] = xstage[:, :].astype(BF16)
            rd = pltpu.make_async_remote_copy(
                src_ref=xsend.at[d - 1],
                dst_ref=xblk.at[3 - d],
                send_sem=xsend_sem.at[d - 1],
                recv_sem=xrecv_sem.at[3 - d],
                device_id=(t,),
                device_id_type=pl.DeviceIdType.MESH,
            )
            rd.start()
            send_rdmas.append(rd)

        cp = pltpu.make_async_copy(
            x_ref.at[pl.ds(me * M, M), :], xstage, stage_sem)
        cp.start()
        cp.wait()
        xblk[3, :, :] = xstage[:, :].astype(BF16)

        def start_w(k_idx, n, buf):
            pltpu.make_async_copy(
                w_ref.at[pl.ds(k_idx * K, K), pl.ds(n * NT, NT)],
                wbuf.at[buf], w_sem.at[buf]).start()

        def wait_w(k_idx, n, buf):
            pltpu.make_async_copy(
                w_ref.at[pl.ds(k_idx * K, K), pl.ds(n * NT, NT)],
                wbuf.at[buf], w_sem.at[buf]).wait()

        def gemm_phase(slot, k_idx, first):
            start_w(k_idx, 0, 0)
            start_w(k_idx, 1, 1)

            def nloop(j, _):
                n0 = 2 * j
                for p in (0, 1):
                    n = n0 + p
                    wait_w(k_idx, n, p)
                    wbf[p, :, :] = wbuf[p, :, :].astype(BF16)

                    @pl.when(n + 2 < N_NT)
                    def _():
                        start_w(k_idx, n + 2, p)

                    t = jnp.dot(xblk[slot], wbf[p],
                                preferred_element_type=F32)
                    sl = pl.ds(pl.multiple_of(n * NT, NT), NT)
                    if first:
                        acc[:, sl] = t
                    else:
                        acc[:, sl] = acc[:, sl] + t
                return 0

            lax.fori_loop(0, N_NT // 2, nloop, 0)

        def wait_recv_slot(q):
            pltpu.make_async_remote_copy(
                src_ref=xsend.at[0], dst_ref=xblk.at[q],
                send_sem=xsend_sem.at[0], recv_sem=xrecv_sem.at[q],
                device_id=(me,), device_id_type=pl.DeviceIdType.MESH,
            ).wait_recv()

        gemm_phase(3, me, first=True)
        wait_recv_slot(0)
        gemm_phase(0, lax.rem(me + 1, N_DEV), first=False)
        wait_recv_slot(2)
        gemm_phase(2, lax.rem(me + 3, N_DEV), first=False)
        wait_recv_slot(1)
        gemm_phase(1, lax.rem(me + 2, N_DEV), first=False)

        def relu_loop(i, amax):
            sl = pl.ds(pl.multiple_of(i * QT, QT), QT)
            v = jnp.maximum(acc[:, sl], 0.0)
            acc[:, sl] = v
            return jnp.maximum(amax, jnp.max(v))

        amax = lax.fori_loop(0, N_QT, relu_loop, jnp.float32(0.0))

        amx_me[:, :] = jnp.full((8, 128), amax, dtype=F32)
        amx_rdmas = []
        for d in (1, 3, 2):
            t = lax.rem(me + d, N_DEV)
            rd = pltpu.make_async_remote_copy(
                src_ref=amx_me,
                dst_ref=amx_peers.at[3 - d],
                send_sem=asend_sem.at[d - 1],
                recv_sem=arecv_sem.at[3 - d],
                device_id=(t,),
                device_id_type=pl.DeviceIdType.MESH,
            )
            rd.start()
            amx_rdmas.append(rd)
        for q in (0, 1, 2):
            pltpu.make_async_remote_copy(
                src_ref=amx_me, dst_ref=amx_peers.at[q],
                send_sem=asend_sem.at[0], recv_sem=arecv_sem.at[q],
                device_id=(me,), device_id_type=pl.DeviceIdType.MESH,
            ).wait_recv()
        gmax = jnp.maximum(amax, jnp.max(amx_peers[:, :, :]))

        scale = gmax / 448.0
        inv = 448.0 / gmax

        def quant_loop(i, _):
            sl = pl.ds(pl.multiple_of(i * QT, QT), QT)
            v = jnp.minimum(acc[:, sl] * inv, 448.0)
            q = v.astype(jnp.float8_e4m3fn).astype(F32)
            acc[:, sl] = q * scale
            return 0

        lax.fori_loop(0, N_QT, quant_loop, 0)

        out_cp = pltpu.make_async_copy(acc, out_ref, out_sem)
        out_cp.start()
        out_cp.wait()

        for rd in send_rdmas:
            rd.wait_send()
        for rd in amx_rdmas:
            rd.wait_send()

    return pl.pallas_call(
        body,
        out_shape=jax.ShapeDtypeStruct((M, N), F32),
        in_specs=[
            pl.BlockSpec(memory_space=pltpu.ANY),
            pl.BlockSpec(memory_space=pltpu.ANY),
        ],
        out_specs=pl.BlockSpec(memory_space=pltpu.ANY),
        scratch_shapes=[
            pltpu.VMEM((M, N), F32),
            pltpu.VMEM((4, M, K), BF16),
            pltpu.VMEM((3, M, K), BF16),
            pltpu.VMEM((M, K), F32),
            pltpu.VMEM((2, K, NT), F32),
            pltpu.VMEM((2, K, NT), BF16),
            pltpu.VMEM((8, 128), F32),
            pltpu.VMEM((3, 8, 128), F32),
            pltpu.SemaphoreType.DMA((3,)),
            pltpu.SemaphoreType.DMA((3,)),
            pltpu.SemaphoreType.DMA((3,)),
            pltpu.SemaphoreType.DMA((3,)),
            pltpu.SemaphoreType.DMA,
            pltpu.SemaphoreType.DMA((2,)),
            pltpu.SemaphoreType.DMA,
        ],
        compiler_params=pltpu.CompilerParams(
            collective_id=0,
            vmem_limit_bytes=64 * 1024 * 1024,
        ),
    )(x, w_mat)


# baseline (device time: 180069 ns/iter reference)
import jax
import jax.numpy as jnp
from jax import lax
from jax.experimental import pallas as pl
from jax.experimental.pallas import tpu as pltpu

N_DEV = 4
M = 1024
K = 1024
N = 8192
NT = 512
N_NT = N // NT
QT = 1024
N_QT = N // QT

F32 = jnp.float32
BF16 = jnp.bfloat16


def kernel(x, w_mat):
    def body(x_ref, w_ref, out_ref, acc, xblk, xsend, xstage, wbuf, wbf,
             amx_me, amx_peers, xsend_sem, xrecv_sem, asend_sem, arecv_sem,
             stage_sem, w_sem, out_sem):
        me = lax.axis_index("i")

        send_rdmas = []
        for d in (1, 3, 2):
            t = lax.rem(me + d, N_DEV)
            cp = pltpu.make_async_copy(
                x_ref.at[pl.ds(t * M, M), :], xstage, stage_sem)
            cp.start()
            cp.wait()
            xsend[d - 1, :, :] = xstage[:, :].astype(BF16)
            rd = pltpu.make_async_remote_copy(
                src_ref=xsend.at[d - 1],
                dst_ref=xblk.at[3 - d],
                send_sem=xsend_sem.at[d - 1],
                recv_sem=xrecv_sem.at[3 - d],
                device_id=(t,),
                device_id_type=pl.DeviceIdType.MESH,
            )
            rd.start()
            send_rdmas.append(rd)

        cp = pltpu.make_async_copy(
            x_ref.at[pl.ds(me * M, M), :], xstage, stage_sem)
        cp.start()
        cp.wait()
        xblk[3, :, :] = xstage[:, :].astype(BF16)

        def start_w(k_idx, n, buf):
            pltpu.make_async_copy(
                w_ref.at[pl.ds(k_idx * K, K), pl.ds(n * NT, NT)],
                wbuf.at[buf], w_sem.at[buf]).start()

        def wait_w(k_idx, n, buf):
            pltpu.make_async_copy(
                w_ref.at[pl.ds(k_idx * K, K), pl.ds(n * NT, NT)],
                wbuf.at[buf], w_sem.at[buf]).wait()

        def gemm_phase(slot, k_idx, first):
            start_w(k_idx, 0, 0)
            start_w(k_idx, 1, 1)

            def nloop(j, _):
                n0 = 2 * j
                for p in (0, 1):
                    n = n0 + p
                    wait_w(k_idx, n, p)
                    wbf[p, :, :] = wbuf[p, :, :].astype(BF16)

                    @pl.when(n + 2 < N_NT)
                    def _():
                        start_w(k_idx, n + 2, p)

                    t = jnp.dot(xblk[slot], wbf[p],
                                preferred_element_type=F32)
                    sl = pl.ds(pl.multiple_of(n * NT, NT), NT)
                    if first:
                        acc[:, sl] = t
                    else:
                        acc[:, sl] = acc[:, sl] + t
                return 0

            lax.fori_loop(0, N_NT // 2, nloop, 0)

        def wait_recv_slot(q):
            pltpu.make_async_remote_copy(
                src_ref=xsend.at[0], dst_ref=xblk.at[q],
                send_sem=xsend_sem.at[0], recv_sem=xrecv_sem.at[q],
                device_id=(me,), device_id_type=pl.DeviceIdType.MESH,
            ).wait_recv()

        gemm_phase(3, me, first=True)
        wait_recv_slot(0)
        gemm_phase(0, lax.rem(me + 1, N_DEV), first=False)
        wait_recv_slot(2)
        gemm_phase(2, lax.rem(me + 3, N_DEV), first=False)
        wait_recv_slot(1)
        gemm_phase(1, lax.rem(me + 2, N_DEV), first=False)

        def relu_loop(i, amax):
            sl = pl.ds(pl.multiple_of(i * QT, QT), QT)
            v = jnp.maximum(acc[:, sl], 0.0)
            acc[:, sl] = v
            return jnp.maximum(amax, jnp.max(v))

        amax = lax.fori_loop(0, N_QT, relu_loop, jnp.float32(0.0))

        amx_me[:, :] = jnp.full((8, 128), amax, dtype=F32)
        amx_rdmas = []
        for d in (1, 3, 2):
            t = lax.rem(me + d, N_DEV)
            rd = pltpu.make_async_remote_copy(
                src_ref=amx_me,
                dst_ref=amx_peers.at[3 - d],
                send_sem=asend_sem.at[d - 1],
                recv_sem=arecv_sem.at[3 - d],
                device_id=(t,),
                device_id_type=pl.DeviceIdType.MESH,
            )
            rd.start()
            amx_rdmas.append(rd)
        for q in (0, 1, 2):
            pltpu.make_async_remote_copy(
                src_ref=amx_me, dst_ref=amx_peers.at[q],
                send_sem=asend_sem.at[0], recv_sem=arecv_sem.at[q],
                device_id=(me,), device_id_type=pl.DeviceIdType.MESH,
            ).wait_recv()
        gmax = jnp.maximum(amax, jnp.max(amx_peers[:, :, :]))

        scale = gmax / 448.0
        inv = 448.0 / gmax

        def quant_loop(i, _):
            sl = pl.ds(pl.multiple_of(i * QT, QT), QT)
            v = jnp.minimum(acc[:, sl] * inv, 448.0)
            q = v.astype(jnp.float8_e4m3fn).astype(F32)
            acc[:, sl] = q * scale
            return 0

        lax.fori_loop(0, N_QT, quant_loop, 0)

        out_cp = pltpu.make_async_copy(acc, out_ref, out_sem)
        out_cp.start()
        out_cp.wait()

        for rd in send_rdmas:
            rd.wait_send()
        for rd in amx_rdmas:
            rd.wait_send()

    return pl.pallas_call(
        body,
        out_shape=jax.ShapeDtypeStruct((M, N), F32),
        in_specs=[
            pl.BlockSpec(memory_space=pl.ANY),
            pl.BlockSpec(memory_space=pl.ANY),
        ],
        out_specs=pl.BlockSpec(memory_space=pl.ANY),
        scratch_shapes=[
            pltpu.VMEM((M, N), F32),
            pltpu.VMEM((4, M, K), BF16),
            pltpu.VMEM((3, M, K), BF16),
            pltpu.VMEM((M, K), F32),
            pltpu.VMEM((2, K, NT), F32),
            pltpu.VMEM((2, K, NT), BF16),
            pltpu.VMEM((8, 128), F32),
            pltpu.VMEM((3, 8, 128), F32),
            pltpu.SemaphoreType.DMA((3,)),
            pltpu.SemaphoreType.DMA((3,)),
            pltpu.SemaphoreType.DMA((3,)),
            pltpu.SemaphoreType.DMA((3,)),
            pltpu.SemaphoreType.DMA,
            pltpu.SemaphoreType.DMA((2,)),
            pltpu.SemaphoreType.DMA,
        ],
        compiler_params=pltpu.CompilerParams(
            vmem_limit_bytes=64 * 1024 * 1024,
        ),
    )(x, w_mat)


# device time: 176751 ns/iter; 1.0188x vs baseline; 1.0188x over previous
import jax
import jax.numpy as jnp
from jax import lax
from jax.experimental import pallas as pl
from jax.experimental.pallas import tpu as pltpu

N_DEV = 4
M = 1024
K = 1024
N = 8192
NT = 1024
N_NT = N // NT
QT = 1024
N_QT = N // QT

F32 = jnp.float32
BF16 = jnp.bfloat16


def kernel(x, w_mat):
    x_bf = x.astype(BF16)

    def body(x_ref, w_ref, out_ref, acc, xblk, wbuf, wbf,
             amx_me, amx_peers, xsend_sem, xrecv_sem, asend_sem, arecv_sem,
             own_sem, w_sem, out_sem):
        me = lax.axis_index("i")

        send_rdmas = []
        for d in (1, 3, 2):
            t = lax.rem(me + d, N_DEV)
            rd = pltpu.make_async_remote_copy(
                src_ref=x_ref.at[pl.ds(t * M, M), :],
                dst_ref=xblk.at[3 - d],
                send_sem=xsend_sem.at[d - 1],
                recv_sem=xrecv_sem.at[3 - d],
                device_id=(t,),
                device_id_type=pl.DeviceIdType.MESH,
            )
            rd.start()
            send_rdmas.append(rd)

        own_cp = pltpu.make_async_copy(
            x_ref.at[pl.ds(me * M, M), :], xblk.at[3], own_sem)
        own_cp.start()

        def start_w(k_idx, n, buf):
            pltpu.make_async_copy(
                w_ref.at[pl.ds(k_idx * K, K), pl.ds(n * NT, NT)],
                wbuf.at[buf], w_sem.at[buf]).start()

        def wait_w(k_idx, n, buf):
            pltpu.make_async_copy(
                w_ref.at[pl.ds(k_idx * K, K), pl.ds(n * NT, NT)],
                wbuf.at[buf], w_sem.at[buf]).wait()

        def gemm_phase(slot, k_idx, amax, first=False, last=False):
            start_w(k_idx, 0, 0)
            start_w(k_idx, 1, 1)
            wait_w(k_idx, 0, 0)
            wbf[0, :, :] = wbuf[0, :, :].astype(BF16)
            start_w(k_idx, 2, 0)
            for n in range(N_NT):
                p = n % 2
                if n + 1 < N_NT:
                    pn = (n + 1) % 2
                    wait_w(k_idx, n + 1, pn)
                    wbf[pn, :, :] = wbuf[pn, :, :].astype(BF16)
                if n + 3 < N_NT:
                    start_w(k_idx, n + 3, (n + 3) % 2)
                t = jnp.dot(xblk[slot], wbf[p], preferred_element_type=F32)
                sl = pl.ds(n * NT, NT)
                if first:
                    acc[:, sl] = t
                elif last:
                    v = jnp.maximum(acc[:, sl] + t, 0.0)
                    acc[:, sl] = v
                    amax = jnp.maximum(amax, jnp.max(v))
                else:
                    acc[:, sl] = acc[:, sl] + t
            return amax

        def wait_recv_slot(q):
            pltpu.make_async_remote_copy(
                src_ref=xblk.at[q], dst_ref=xblk.at[q],
                send_sem=xsend_sem.at[0], recv_sem=xrecv_sem.at[q],
                device_id=(me,), device_id_type=pl.DeviceIdType.MESH,
            ).wait_recv()

        amax = jnp.float32(0.0)
        own_cp.wait()
        amax = gemm_phase(3, me, amax, first=True)
        wait_recv_slot(0)
        amax = gemm_phase(0, lax.rem(me + 1, N_DEV), amax)
        wait_recv_slot(2)
        amax = gemm_phase(2, lax.rem(me + 3, N_DEV), amax)
        wait_recv_slot(1)
        amax = gemm_phase(1, lax.rem(me + 2, N_DEV), amax, last=True)

        amx_me[:, :] = jnp.full((8, 128), amax, dtype=F32)
        amx_rdmas = []
        for d in (1, 3, 2):
            t = lax.rem(me + d, N_DEV)
            rd = pltpu.make_async_remote_copy(
                src_ref=amx_me,
                dst_ref=amx_peers.at[3 - d],
                send_sem=asend_sem.at[d - 1],
                recv_sem=arecv_sem.at[3 - d],
                device_id=(t,),
                device_id_type=pl.DeviceIdType.MESH,
            )
            rd.start()
            amx_rdmas.append(rd)
        for q in (0, 1, 2):
            pltpu.make_async_remote_copy(
                src_ref=amx_me, dst_ref=amx_peers.at[q],
                send_sem=asend_sem.at[0], recv_sem=arecv_sem.at[q],
                device_id=(me,), device_id_type=pl.DeviceIdType.MESH,
            ).wait_recv()
        gmax = jnp.maximum(amax, jnp.max(amx_peers[:, :, :]))

        scale = gmax / 448.0
        inv = 448.0 / gmax

        out_cps = []
        for i in range(N_QT):
            sl = pl.ds(i * QT, QT)
            v = jnp.minimum(acc[:, sl] * inv, 448.0)
            q = v.astype(jnp.float8_e4m3fn).astype(F32)
            acc[:, sl] = q * scale
            if i >= 2:
                out_cps[i - 2].wait()
            cp = pltpu.make_async_copy(
                acc.at[:, sl], out_ref.at[:, sl], out_sem.at[i % 2])
            cp.start()
            out_cps.append(cp)
        out_cps[-2].wait()
        out_cps[-1].wait()

        for rd in send_rdmas:
            rd.wait_send()
        for rd in amx_rdmas:
            rd.wait_send()

    return pl.pallas_call(
        body,
        out_shape=jax.ShapeDtypeStruct((M, N), F32),
        in_specs=[
            pl.BlockSpec(memory_space=pl.ANY),
            pl.BlockSpec(memory_space=pl.ANY),
        ],
        out_specs=pl.BlockSpec(memory_space=pl.ANY),
        scratch_shapes=[
            pltpu.VMEM((M, N), F32),
            pltpu.VMEM((4, M, K), BF16),
            pltpu.VMEM((2, K, NT), F32),
            pltpu.VMEM((2, K, NT), BF16),
            pltpu.VMEM((8, 128), F32),
            pltpu.VMEM((3, 8, 128), F32),
            pltpu.SemaphoreType.DMA((3,)),
            pltpu.SemaphoreType.DMA((3,)),
            pltpu.SemaphoreType.DMA((3,)),
            pltpu.SemaphoreType.DMA((3,)),
            pltpu.SemaphoreType.DMA,
            pltpu.SemaphoreType.DMA((2,)),
            pltpu.SemaphoreType.DMA((2,)),
        ],
        compiler_params=pltpu.CompilerParams(
            vmem_limit_bytes=64 * 1024 * 1024,
        ),
    )(x_bf, w_mat)


# device time: 172418 ns/iter; 1.0444x vs baseline; 1.0251x over previous
import os

import jax
import jax.numpy as jnp
from jax import lax
from jax.experimental import pallas as pl
from jax.experimental.pallas import tpu as pltpu

N_DEV = 4
M = 1024
K = 1024
N = 8192
NT = 1024
N_NT = N // NT
NSTEP = N_DEV * N_NT

F32 = jnp.float32
BF16 = jnp.bfloat16

SLOT_ORDER = (3, 0, 2, 1)


def kernel(x, w_mat):
    x_bf = x.astype(BF16)

    def body(x_ref, w_ref, out_ref, xblk, wbuf, wbf,
             amx_me, amx_peers, xsend_sem, xrecv_sem, asend_sem, arecv_sem,
             own_sem, w_sem):
        me = lax.axis_index("i")
        acc = out_ref

        send_rdmas = []
        for d in (1, 3, 2):
            t = lax.rem(me + d, N_DEV)
            rd = pltpu.make_async_remote_copy(
                src_ref=x_ref.at[pl.ds(t * M, M), :],
                dst_ref=xblk.at[3 - d],
                send_sem=xsend_sem.at[d - 1],
                recv_sem=xrecv_sem.at[3 - d],
                device_id=(t,),
                device_id_type=pl.DeviceIdType.MESH,
            )
            rd.start()
            send_rdmas.append(rd)

        own_cp = pltpu.make_async_copy(
            x_ref.at[pl.ds(me * M, M), :], xblk.at[3], own_sem)
        own_cp.start()

        k_of_phase = [me,
                      lax.rem(me + 1, N_DEV),
                      lax.rem(me + 3, N_DEV),
                      lax.rem(me + 2, N_DEV)]

        def start_w(s):
            k_idx, n = k_of_phase[s // N_NT], s % N_NT
            pltpu.make_async_copy(
                w_ref.at[pl.ds(k_idx * K, K), pl.ds(n * NT, NT)],
                wbuf.at[s % 4], w_sem.at[s % 4]).start()

        def wait_w(s):
            pltpu.make_async_copy(
                w_ref.at[pl.ds(0, K), pl.ds(0, NT)],
                wbuf.at[s % 4], w_sem.at[s % 4]).wait()

        def wait_recv_slot(q):
            pltpu.make_async_remote_copy(
                src_ref=xblk.at[q], dst_ref=xblk.at[q],
                send_sem=xsend_sem.at[0], recv_sem=xrecv_sem.at[q],
                device_id=(me,), device_id_type=pl.DeviceIdType.MESH,
            ).wait_recv()

        for s in range(4):
            start_w(s)
        wait_w(0)
        wbf[0, :, :] = wbuf[0, :, :].astype(BF16)
        start_w(4)
        own_cp.wait()

        amax = jnp.float32(0.0)
        for s in range(NSTEP):
            ph, n = s // N_NT, s % N_NT
            if s + 1 < NSTEP:
                wait_w(s + 1)
                wbf[(s + 1) % 2, :, :] = wbuf[(s + 1) % 4, :, :].astype(BF16)
            if s + 5 < NSTEP:
                start_w(s + 5)
            if n == 0 and ph > 0:
                wait_recv_slot(SLOT_ORDER[ph])
            t = jnp.dot(xblk[SLOT_ORDER[ph]], wbf[s % 2],
                        preferred_element_type=F32)
            sl = pl.ds(n * NT, NT)
            if ph == 0:
                acc[:, sl] = t
            elif ph == 3:
                v = jnp.maximum(acc[:, sl] + t, 0.0)
                acc[:, sl] = v
                amax = jnp.maximum(amax, jnp.max(v))
            else:
                acc[:, sl] = acc[:, sl] + t

        amx_me[:, :] = jnp.full((8, 128), amax, dtype=F32)
        amx_rdmas = []
        for d in (1, 3, 2):
            t = lax.rem(me + d, N_DEV)
            rd = pltpu.make_async_remote_copy(
                src_ref=amx_me,
                dst_ref=amx_peers.at[3 - d],
                send_sem=asend_sem.at[d - 1],
                recv_sem=arecv_sem.at[3 - d],
                device_id=(t,),
                device_id_type=pl.DeviceIdType.MESH,
            )
            rd.start()
            amx_rdmas.append(rd)
        for q in (0, 1, 2):
            pltpu.make_async_remote_copy(
                src_ref=amx_me, dst_ref=amx_peers.at[q],
                send_sem=asend_sem.at[0], recv_sem=arecv_sem.at[q],
                device_id=(me,), device_id_type=pl.DeviceIdType.MESH,
            ).wait_recv()
        gmax = jnp.maximum(amax, jnp.max(amx_peers[:, :, :]))

        scale = gmax / 448.0
        inv = 448.0 / gmax
        for i in range(N_NT):
            sl = pl.ds(i * NT, NT)
            q = (acc[:, sl] * inv).astype(jnp.float8_e4m3fn).astype(F32)
            acc[:, sl] = q * scale

        for rd in send_rdmas:
            rd.wait_send()
        for rd in amx_rdmas:
            rd.wait_send()

    return pl.pallas_call(
        body,
        out_shape=jax.ShapeDtypeStruct((M, N), F32),
        in_specs=[
            pl.BlockSpec(memory_space=pl.ANY),
            pl.BlockSpec(memory_space=pl.ANY),
        ],
        out_specs=pl.BlockSpec(memory_space=pltpu.MemorySpace.VMEM),
        scratch_shapes=[
            pltpu.VMEM((4, M, K), BF16),
            pltpu.VMEM((4, K, NT), F32),
            pltpu.VMEM((2, K, NT), BF16),
            pltpu.VMEM((8, 128), F32),
            pltpu.VMEM((3, 8, 128), F32),
            pltpu.SemaphoreType.DMA((3,)),
            pltpu.SemaphoreType.DMA((3,)),
            pltpu.SemaphoreType.DMA((3,)),
            pltpu.SemaphoreType.DMA((3,)),
            pltpu.SemaphoreType.DMA,
            pltpu.SemaphoreType.DMA((4,)),
        ],
        compiler_params=pltpu.CompilerParams(
            vmem_limit_bytes=64 * 1024 * 1024,
        ),
    )(x_bf, w_mat)


# device time: 172268 ns/iter; 1.0453x vs baseline; 1.0009x over previous
import os

import jax
import jax.numpy as jnp
from jax import lax
from jax.experimental import pallas as pl
from jax.experimental.pallas import tpu as pltpu

N_DEV = 4
M = 1024
K = 1024
N = 8192
NT = 1024
N_NT = N // NT
NSTEP = N_DEV * N_NT

F32 = jnp.float32
BF16 = jnp.bfloat16

SLOT_ORDER = (3, 0, 2, 1)


def kernel(x, w_mat):
    x_bf = x.astype(BF16)

    def body(x_ref, w_ref, out_ref, xblk, wbuf, wbf,
             amx_me, amx_peers, xsend_sem, xrecv_sem, asend_sem, arecv_sem,
             own_sem, w_sem):
        me = lax.axis_index("i")
        acc = out_ref

        send_rdmas = []
        for d in (1, 3, 2):
            t = lax.rem(me + d, N_DEV)
            rd = pltpu.make_async_remote_copy(
                src_ref=x_ref.at[pl.ds(t * M, M), :],
                dst_ref=xblk.at[3 - d],
                send_sem=xsend_sem.at[d - 1],
                recv_sem=xrecv_sem.at[3 - d],
                device_id=(t,),
                device_id_type=pl.DeviceIdType.MESH,
            )
            rd.start()
            send_rdmas.append(rd)

        own_cp = pltpu.make_async_copy(
            x_ref.at[pl.ds(me * M, M), :], xblk.at[3], own_sem)
        own_cp.start()

        k_of_phase = [me,
                      lax.rem(me + 1, N_DEV),
                      lax.rem(me + 3, N_DEV),
                      lax.rem(me + 2, N_DEV)]

        def start_w(s):
            k_idx, n = k_of_phase[s // N_NT], s % N_NT
            pltpu.make_async_copy(
                w_ref.at[pl.ds(k_idx * K, K), pl.ds(n * NT, NT)],
                wbuf.at[s % 2], w_sem.at[s % 2]).start()

        def wait_w(s):
            pltpu.make_async_copy(
                w_ref.at[pl.ds(0, K), pl.ds(0, NT)],
                wbuf.at[s % 2], w_sem.at[s % 2]).wait()

        def wait_recv_slot(q):
            pltpu.make_async_remote_copy(
                src_ref=xblk.at[q], dst_ref=xblk.at[q],
                send_sem=xsend_sem.at[0], recv_sem=xrecv_sem.at[q],
                device_id=(me,), device_id_type=pl.DeviceIdType.MESH,
            ).wait_recv()

        start_w(0)
        start_w(1)
        wait_w(0)
        wbf[0, :, :] = wbuf[0, :, :].astype(BF16)
        start_w(2)
        own_cp.wait()

        amax = jnp.float32(0.0)
        for s in range(NSTEP):
            ph, n = s // N_NT, s % N_NT
            if s + 1 < NSTEP:
                wait_w(s + 1)
                wbf[(s + 1) % 2, :, :] = wbuf[(s + 1) % 2, :, :].astype(BF16)
            if s + 3 < NSTEP:
                start_w(s + 3)
            if n == 0 and ph > 0:
                wait_recv_slot(SLOT_ORDER[ph])
            t = jnp.dot(xblk[SLOT_ORDER[ph]], wbf[s % 2],
                        preferred_element_type=F32)
            sl = pl.ds(n * NT, NT)
            if ph == 0:
                acc[:, sl] = t
            elif ph == 3:
                v = jnp.maximum(acc[:, sl] + t, 0.0)
                acc[:, sl] = v
                amax = jnp.maximum(amax, jnp.max(v))
            else:
                acc[:, sl] = acc[:, sl] + t

        amx_me[:, :] = jnp.full((8, 128), amax, dtype=F32)
        amx_rdmas = []
        for d in (1, 3, 2):
            t = lax.rem(me + d, N_DEV)
            rd = pltpu.make_async_remote_copy(
                src_ref=amx_me,
                dst_ref=amx_peers.at[3 - d],
                send_sem=asend_sem.at[d - 1],
                recv_sem=arecv_sem.at[3 - d],
                device_id=(t,),
                device_id_type=pl.DeviceIdType.MESH,
            )
            rd.start()
            amx_rdmas.append(rd)
        for q in (0, 1, 2):
            pltpu.make_async_remote_copy(
                src_ref=amx_me, dst_ref=amx_peers.at[q],
                send_sem=asend_sem.at[0], recv_sem=arecv_sem.at[q],
                device_id=(me,), device_id_type=pl.DeviceIdType.MESH,
            ).wait_recv()
        gmax = jnp.maximum(amax, jnp.max(amx_peers[:, :, :]))

        scale = gmax / 448.0
        inv = 448.0 / gmax
        for i in range(N_NT):
            sl = pl.ds(i * NT, NT)
            q = (acc[:, sl] * inv).astype(jnp.float8_e4m3fn).astype(F32)
            acc[:, sl] = q * scale

        for rd in send_rdmas:
            rd.wait_send()
        for rd in amx_rdmas:
            rd.wait_send()

    return pl.pallas_call(
        body,
        out_shape=jax.ShapeDtypeStruct((M, N), F32),
        in_specs=[
            pl.BlockSpec(memory_space=pl.ANY),
            pl.BlockSpec(memory_space=pl.ANY),
        ],
        out_specs=pl.BlockSpec(memory_space=pltpu.MemorySpace.VMEM),
        scratch_shapes=[
            pltpu.VMEM((4, M, K), BF16),
            pltpu.VMEM((2, K, NT), F32),
            pltpu.VMEM((2, K, NT), BF16),
            pltpu.VMEM((8, 128), F32),
            pltpu.VMEM((3, 8, 128), F32),
            pltpu.SemaphoreType.DMA((3,)),
            pltpu.SemaphoreType.DMA((3,)),
            pltpu.SemaphoreType.DMA((3,)),
            pltpu.SemaphoreType.DMA((3,)),
            pltpu.SemaphoreType.DMA,
            pltpu.SemaphoreType.DMA((2,)),
        ],
        compiler_params=pltpu.CompilerParams(
            vmem_limit_bytes=64 * 1024 * 1024,
        ),
    )(x_bf, w_mat)
